# Initial kernel scaffold; baseline (speedup 1.0000x reference)
import jax
import jax.numpy as jnp
from jax.experimental import pallas as pl


def kernel(input_ids, base_table, lottie_table):
    raise NotImplementedError("write your pallas kernel here")


# speedup vs baseline: 2.1462x; 2.1462x over previous
import functools

import jax
import jax.numpy as jnp
from jax import lax
from jax.experimental import pallas as pl
from jax.experimental.pallas import tpu as pltpu
from jax.experimental.pallas import tpu_sc as plsc

_NC = 2
_NS = 16
_NW = _NC * _NS
_LANES = 16

_CHUNK = 512
_GPIECE = 128


def _build(N, V, NNEW, H):
    per_w = N // _NW
    n_chunks = per_w // _CHUNK
    n_grp = _CHUNK // _LANES
    n_piece = _CHUNK // _GPIECE

    mesh = plsc.VectorSubcoreMesh(
        core_axis_name="c", subcore_axis_name="s",
        num_cores=_NC, num_subcores=_NS)

    @functools.partial(
        pl.kernel,
        out_type=jax.ShapeDtypeStruct((N, H), jnp.float32),
        mesh=mesh,
        compiler_params=pltpu.CompilerParams(
            use_tc_tiling_on_sc=False, needs_layout_passes=False),
        scratch_types=[
            pltpu.VMEM((_CHUNK,), jnp.int32),
            pltpu.VMEM((_CHUNK,), jnp.int32),
            pltpu.VMEM((_CHUNK + _LANES,), jnp.int32),
            pltpu.VMEM((_CHUNK + _LANES,), jnp.int32),
            pltpu.VMEM((_CHUNK, H), jnp.float32),
            pltpu.VMEM((_CHUNK, H), jnp.float32),
            pltpu.SemaphoreType.DMA,
            pltpu.SemaphoreType.DMA,
        ],
    )
    def k(ids_hbm, base_hbm, lottie_hbm, out_hbm,
          ids_v, bidx_v, lidx_v, pos_v, rows_v, lrows_v, sem, lsem):
        wid = lax.axis_index("s") * _NC + lax.axis_index("c")
        base0 = wid * per_w

        def chunk_body(ci, _):
            off = base0 + ci * _CHUNK
            pltpu.sync_copy(ids_hbm.at[pl.ds(off, _CHUNK)], ids_v)

            def grp(g, c):
                ids16 = ids_v[pl.ds(g * _LANES, _LANES)]
                m = ids16 >= V
                bidx_v[pl.ds(g * _LANES, _LANES)] = jnp.minimum(ids16, V - 1)
                incl = plsc.cumsum(m.astype(jnp.int32))
                dstv = c + incl - 1
                plsc.store_scatter(lidx_v, [dstv], ids16 - V, mask=m)
                posv = lax.iota(jnp.int32, _LANES) + g * _LANES
                plsc.store_scatter(pos_v, [dstv], posv, mask=m)
                return c + jnp.sum(m.astype(jnp.int32))

            c = lax.fori_loop(0, n_grp, grp, jnp.int32(0))
            lidx_v[pl.ds(c, _LANES)] = jnp.zeros((_LANES,), jnp.int32)

            cps = []
            for p in range(n_piece):
                cps.append(pltpu.async_copy(
                    base_hbm.at[bidx_v.at[pl.ds(p * _GPIECE, _GPIECE)]],
                    rows_v.at[pl.ds(p * _GPIECE, _GPIECE)],
                    sem))
            for cp in cps:
                cp.wait()

            def lgather(t, _):
                pltpu.async_copy(
                    lottie_hbm.at[lidx_v.at[pl.ds(t * _LANES, _LANES)]],
                    lrows_v.at[pl.ds(t * _LANES, _LANES)],
                    lsem).wait()
                return 0

            lax.fori_loop(0, (c + _LANES - 1) // _LANES, lgather, 0)

            def cmb(g2, _):
                j0 = g2 // 4
                colstart = (g2 % 4) * _LANES
                jv = jnp.full((_LANES,), j0, jnp.int32)
                colv = colstart + lax.iota(jnp.int32, _LANES)
                val = plsc.load_gather(lrows_v, [jv, colv])
                posv = plsc.load_gather(pos_v, [jv])
                plsc.store_scatter(rows_v, [posv, colv], val)
                return 0

            lax.fori_loop(0, c * (H // _LANES), cmb, 0)

            pltpu.sync_copy(rows_v, out_hbm.at[pl.ds(off, _CHUNK)])
            return 0

        lax.fori_loop(0, n_chunks, chunk_body, 0)

    return k


def kernel(input_ids, base_table, lottie_table):
    V, H = base_table.shape
    NNEW = lottie_table.shape[0]
    ids = input_ids.reshape(-1)
    N = ids.shape[0]
    k = _build(N, V, NNEW, H)
    out = k(ids, base_table, lottie_table)
    return out.reshape(input_ids.shape + (H,))
